# baseline (speedup 1.0000x reference)
import functools

import jax
import jax.numpy as jnp
from jax import lax
from jax.experimental import pallas as pl
from jax.experimental.pallas import tpu as pltpu
from jax.experimental.pallas import tpu_sc as plsc

_NJ, _NG, _C = 50000, 25000, 64
_NJP, _NGP = 50176, 25088
_CH = 512
_NCH_JJ = 1600
_NCH_GJ = 320

_GATES = ("i", "f", "g", "o")
_B = 512
_F32 = jnp.float32
_BF16 = jnp.bfloat16

_MESH = dict(core_axis_name="c", subcore_axis_name="s")
_SC_PARAMS = pltpu.CompilerParams(use_tc_tiling_on_sc=False)


def _sc_raw_agg(table, eidx, zrows, n_dst, nch):
    per_core = nch // 2
    per_sub = per_core // 16
    rps = n_dst // 16

    @functools.partial(
        pl.kernel,
        out_type=jax.ShapeDtypeStruct((2, n_dst, 16), _F32),
        mesh=plsc.VectorSubcoreMesh(**_MESH),
        scratch_types=[
            pltpu.VMEM((1, 2, _CH), jnp.int32),
            pltpu.VMEM((_CH, 16), _F32),
            pltpu.VMEM_SHARED((n_dst, 16), _F32),
            pltpu.SemaphoreType.DMA,
        ],
        compiler_params=_SC_PARAMS,
    )
    def k(table_h, eidx_h, z_h, out_h, ebuf, rows, acc, sem):
        cid = lax.axis_index("c")
        sid = lax.axis_index("s")
        r0 = sid * rps
        pltpu.sync_copy(z_h.at[pl.ds(0, rps)], acc.at[pl.ds(r0, rps)])
        plsc.subcore_barrier()
        c0 = cid * per_core + sid * per_sub

        @pl.loop(0, per_sub)
        def _(i):
            pltpu.sync_copy(eidx_h.at[pl.ds(c0 + i, 1)], ebuf)
            pltpu.async_copy(table_h.at[ebuf.at[0, 0]], rows, sem).wait()
            pltpu.sync_copy(rows, acc.at[ebuf.at[0, 1]], add=True)

        plsc.subcore_barrier()

        @pl.when(cid == 0)
        def _():
            pltpu.sync_copy(acc.at[pl.ds(r0, rps)], out_h.at[0].at[pl.ds(r0, rps)])

        @pl.when(cid == 1)
        def _():
            pltpu.sync_copy(acc.at[pl.ds(r0, rps)], out_h.at[1].at[pl.ds(r0, rps)])

    return k(table, eidx, zrows)


def _sc_h_comb(tj0, tj1, tg0, tg1, e_jj, e_gj, zrows):
    rps = _NJP // 16

    @functools.partial(
        pl.kernel,
        out_type=[jax.ShapeDtypeStruct((_NJP, 64), _BF16)] * 4,
        mesh=plsc.VectorSubcoreMesh(**_MESH),
        scratch_types=[
            pltpu.VMEM((1, 2, _CH), jnp.int32),
            pltpu.VMEM((_CH, 64), _BF16),
            pltpu.VMEM_SHARED((_NJP, 64), _BF16),
            pltpu.SemaphoreType.DMA,
        ],
        compiler_params=_SC_PARAMS,
    )
    def k(tj0_h, tj1_h, tg0_h, tg1_h, ejj_h, egj_h, z_h,
          ojj0_h, ojj1_h, ogj0_h, ogj1_h, ebuf, rows, acc, sem):
        cid = lax.axis_index("c")
        sid = lax.axis_index("s")
        r0 = sid * rps

        def phase(t0_h, t1_h, eidx_h, nch, o0_h, o1_h):
            per_sub = nch // 16
            c0 = sid * per_sub
            pltpu.sync_copy(z_h.at[pl.ds(0, rps)], acc.at[pl.ds(r0, rps)])
            plsc.subcore_barrier()

            def body(t_h):
                @pl.loop(0, per_sub)
                def _(i):
                    pltpu.sync_copy(eidx_h.at[pl.ds(c0 + i, 1)], ebuf)
                    pltpu.async_copy(t_h.at[ebuf.at[0, 0]], rows, sem).wait()
                    pltpu.sync_copy(rows, acc.at[ebuf.at[0, 1]], add=True)

            @pl.when(cid == 0)
            def _():
                body(t0_h)

            @pl.when(cid == 1)
            def _():
                body(t1_h)

            plsc.subcore_barrier()

            @pl.when(cid == 0)
            def _():
                pltpu.sync_copy(acc.at[pl.ds(r0, rps)], o0_h.at[pl.ds(r0, rps)])

            @pl.when(cid == 1)
            def _():
                pltpu.sync_copy(acc.at[pl.ds(r0, rps)], o1_h.at[pl.ds(r0, rps)])

        phase(tj0_h, tj1_h, ejj_h, _NCH_JJ, ojj0_h, ojj1_h)
        phase(tg0_h, tg1_h, egj_h, _NCH_GJ, ogj0_h, ogj1_h)

    return k(tj0, tj1, tg0, tg1, e_jj, e_gj, zrows)


def _dot(a, b):
    return lax.dot_general(a, b, (((1,), (0,)), ((), ())),
                           precision=lax.Precision.HIGHEST,
                           preferred_element_type=_F32)


def _lstm(z, c_prev):
    ig = jax.nn.sigmoid(z[:, :64])
    fg = jax.nn.sigmoid(z[:, 64:128])
    gg = jnp.tanh(z[:, 128:192])
    og = jax.nn.sigmoid(z[:, 192:256])
    c = ig * gg if c_prev is None else fg * c_prev + ig * gg
    return og * jnp.tanh(c), c


def _mean16(p0, p1, col):
    s = p0 + p1
    return s * (1.0 / jnp.maximum(s[:, col:col + 1], 1.0))


def _split_dot(a0, a1, w):
    return _dot(a0, w[:32]) + _dot(a1, w[32:])


def _blk(width):
    return pl.BlockSpec((_B, width), lambda i: (i, 0))


def _full(a):
    return pl.BlockSpec(a.shape, lambda i: (0, 0))


def _tc_cell0(n_pad, xp, aggs, cols, weights):
    grid = (n_pad // _B,)
    na = len(aggs)
    arrays = [xp] + [p for a in aggs for p in a] + list(weights)
    specs = ([_blk(16)] * (1 + 2 * na) + [_full(w) for w in weights])

    def body(*refs):
        xr = refs[0]
        ar = refs[1:1 + 2 * na]
        wr = refs[1 + 2 * na:-2]
        t0o, t1o = refs[-2], refs[-1]
        x = xr[...]
        means = [_mean16(ar[2 * i][...], ar[2 * i + 1][...], cols[i])
                 for i in range(na)]
        it = iter(wr)
        ze = _dot(x, next(it)[...])
        for m in means:
            ze = ze + _dot(m, next(it)[...])
        ze = ze + next(it)[...]
        hE, cE = _lstm(ze, None)
        zd = _dot(x, next(it)[...]) + _dot(hE, next(it)[...])
        for m in means:
            zd = zd + _dot(m, next(it)[...])
        zd = zd + next(it)[...]
        hD, _ = _lstm(zd, cE)
        t0o[...] = jnp.concatenate([hE[:, :32], hD[:, :32]], 1).astype(_BF16)
        t1o[...] = jnp.concatenate([hE[:, 32:], hD[:, 32:]], 1).astype(_BF16)

    out_shape = [jax.ShapeDtypeStruct((n_pad, 64), _BF16)] * 2
    out_specs = [_blk(64)] * 2
    return pl.pallas_call(body, grid=grid, in_specs=specs,
                          out_specs=out_specs, out_shape=out_shape)(*arrays)


def _tc_final(tj0, tj1, jjA, gjA, rjj, rgj, xp, weights, whead):
    grid = (_NJP // _B,)
    arrays = ([tj0, tj1, jjA[0], jjA[1], gjA[0], gjA[1],
               rjj[0], rjj[1], rgj[0], rgj[1], xp]
              + list(weights) + [whead])
    specs = ([_blk(64)] * 6 + [_blk(16)] * 5
             + [_full(w) for w in weights] + [_full(whead)])

    def body(tj0r, tj1r, jjA0r, jjA1r, gjA0r, gjA1r,
             rjj0r, rjj1r, rgj0r, rgj1r, xr,
             we_x, we_jj, we_gj, be, wd_x, wd_h, wd_jj, wd_gj, bd,
             whr, oref):
        t0 = tj0r[...].astype(_F32)
        t1 = tj1r[...].astype(_F32)
        jA0 = jjA0r[...].astype(_F32)
        jA1 = jjA1r[...].astype(_F32)
        gA0 = gjA0r[...].astype(_F32)
        gA1 = gjA1r[...].astype(_F32)
        x = xr[...]
        sjj = rjj0r[...] + rjj1r[...]
        r_jj = 1.0 / jnp.maximum(sjj[:, 5:6], 1.0)
        sgj = rgj0r[...] + rgj1r[...]
        r_gj = 1.0 / jnp.maximum(sgj[:, 8:9], 1.0)

        ze = (_split_dot(t0[:, :32], t1[:, :32], we_x[...])
              + _split_dot(jA0[:, :32] * r_jj, jA1[:, :32] * r_jj, we_jj[...])
              + _split_dot(gA0[:, :32] * r_gj, gA1[:, :32] * r_gj, we_gj[...])
              + be[...])
        hE, cE = _lstm(ze, None)
        zd = (_split_dot(t0[:, 32:], t1[:, 32:], wd_x[...])
              + _dot(hE, wd_h[...])
              + _split_dot(jA0[:, 32:] * r_jj, jA1[:, 32:] * r_jj, wd_jj[...])
              + _split_dot(gA0[:, 32:] * r_gj, gA1[:, 32:] * r_gj, wd_gj[...])
              + bd[...])
        hD, cD = _lstm(zd, cE)
        wh = whr[...]
        zh = _dot(hD, wh[:64]) + _dot(cD, wh[64:128]) + _dot(x, wh[128:144])
        lane = lax.broadcasted_iota(jnp.int32, zh.shape, 1)
        full = jnp.where(lane < 2, jnp.tanh(zh) / 5.0, jax.nn.sigmoid(zh))
        oref[...] = full[:, :3]

    out_shape = jax.ShapeDtypeStruct((_NJP, 3), _F32)
    out_specs = pl.BlockSpec((_B, 3), lambda i: (i, 0))
    return pl.pallas_call(body, grid=grid, in_specs=specs,
                          out_specs=out_specs, out_shape=out_shape)(*arrays)


def _prep_edges(ei, nch, dst_pad):
    e = ei.astype(jnp.int32)
    pad = nch * _CH - e.shape[1]
    src = jnp.concatenate([e[0], jnp.zeros((pad,), jnp.int32)])
    dst = jnp.concatenate([e[1], jnp.full((pad,), dst_pad, jnp.int32)])
    return jnp.stack([src.reshape(nch, _CH), dst.reshape(nch, _CH)], axis=1)


def kernel(x_joint, x_grain, params, edge_jj, edge_gj, edge_jg, edge_attr_jj):
    p = params

    def wcat(stem, tail, pad_to=None):
        w = jnp.concatenate([p[f"{stem}_{g}_{tail}"] for g in _GATES], 1)
        if pad_to is not None and w.shape[0] < pad_to:
            w = jnp.zeros((pad_to, w.shape[1]), _F32).at[:w.shape[0]].set(w)
        return w

    def bcat(pre, nt):
        return jnp.concatenate(
            [p[f"{pre}_b_{g}_{nt}"] for g in _GATES]).reshape(1, 256)

    joint_pad = (jnp.zeros((_NJP, 16), _F32)
                 .at[:_NJ, :5].set(x_joint).at[:_NJ, 5].set(1.0))
    grain_pad = (jnp.zeros((_NGP, 16), _F32)
                 .at[:_NG, :8].set(x_grain).at[:_NG, 8].set(1.0))

    e_jj = _prep_edges(edge_jj, _NCH_JJ, _NJP - 1)
    e_gj = _prep_edges(edge_gj, _NCH_GJ, _NJP - 1)
    e_jg = _prep_edges(edge_jg, _NCH_GJ, _NGP - 1)

    z16 = jnp.zeros((_NJP // 16, 16), _F32)
    z64 = jnp.zeros((_NJP // 16, 64), _BF16)

    raw_jj = _sc_raw_agg(joint_pad, e_jj, z16, _NJP, _NCH_JJ)
    raw_gj = _sc_raw_agg(grain_pad, e_gj, z16, _NJP, _NCH_GJ)
    raw_jg = _sc_raw_agg(joint_pad, e_jg, z16[:_NGP // 16], _NGP, _NCH_GJ)
    rjj = [raw_jj[0], raw_jj[1]]
    rgj = [raw_gj[0], raw_gj[1]]
    rjg = [raw_jg[0], raw_jg[1]]

    tj0, tj1 = _tc_cell0(
        _NJP, joint_pad, [rjj, rgj], [5, 8],
        (wcat("enc0_Wx", "joint", 16),
         wcat("enc0_We", "jj", 16), wcat("enc0_We", "gj", 16),
         bcat("enc0", "joint"),
         wcat("dec0_Wx", "joint", 16), wcat("dec0_Wh", "joint"),
         wcat("dec0_We", "jj", 16), wcat("dec0_We", "gj", 16),
         bcat("dec0", "joint")))
    tg0, tg1 = _tc_cell0(
        _NGP, grain_pad, [rjg], [5],
        (wcat("enc0_Wx", "grain", 16),
         wcat("enc0_We", "jg", 16),
         bcat("enc0", "grain"),
         wcat("dec0_Wx", "grain", 16), wcat("dec0_Wh", "grain"),
         wcat("dec0_We", "jg", 16),
         bcat("dec0", "grain")))

    jjA0, jjA1, gjA0, gjA1 = _sc_h_comb(tj0, tj1, tg0, tg1, e_jj, e_gj, z64)

    whead = jnp.zeros((144, 128), _F32)
    whead = whead.at[:129, 0:2].set(p["lin1_W"])
    whead = whead.at[:129, 2:3].set(p["lin2_W"])
    whead = whead.at[133, 0:2].set(p["lin1_b"])
    whead = whead.at[133, 2].set(p["lin2_b"][0])

    out = _tc_final(
        tj0, tj1, [jjA0, jjA1], [gjA0, gjA1], rjj, rgj, joint_pad,
        (wcat("enc1_Wx", "joint"),
         wcat("enc1_We", "jj"), wcat("enc1_We", "gj"),
         bcat("enc1", "joint"),
         wcat("dec1_Wx", "joint"), wcat("dec1_Wh", "joint"),
         wcat("dec1_We", "jj"), wcat("dec1_We", "gj"),
         bcat("dec1", "joint")),
        whead)

    return out[:_NJ]


# speedup vs baseline: 1.5554x; 1.5554x over previous
import functools

import jax
import jax.numpy as jnp
from jax import lax
from jax.experimental import pallas as pl
from jax.experimental.pallas import tpu as pltpu
from jax.experimental.pallas import tpu_sc as plsc

_NJ, _NG, _C = 50000, 25000, 64
_NJP, _NGP = 50176, 25088
_CH = 512
_NCH_JJ = 1600
_NCH_GJ = 320

_GATES = ("i", "f", "g", "o")
_B = 512
_F32 = jnp.float32
_BF16 = jnp.bfloat16

_MESH = dict(core_axis_name="c", subcore_axis_name="s")
_SC_PARAMS = pltpu.CompilerParams(use_tc_tiling_on_sc=False)


def _sc_raw_agg(table, eidx, zrows, n_dst, nch):
    per_core = nch // 2
    per_sub = per_core // 16
    rps = n_dst // 16

    @functools.partial(
        pl.kernel,
        out_type=jax.ShapeDtypeStruct((2, n_dst, 16), _F32),
        mesh=plsc.VectorSubcoreMesh(**_MESH),
        scratch_types=[
            pltpu.VMEM((1, 2, _CH), jnp.int32),
            pltpu.VMEM((_CH, 16), _F32),
            pltpu.VMEM_SHARED((n_dst, 16), _F32),
            pltpu.SemaphoreType.DMA,
        ],
        compiler_params=_SC_PARAMS,
    )
    def k(table_h, eidx_h, z_h, out_h, ebuf, rows, acc, sem):
        cid = lax.axis_index("c")
        sid = lax.axis_index("s")
        r0 = sid * rps
        pltpu.sync_copy(z_h.at[pl.ds(0, rps)], acc.at[pl.ds(r0, rps)])
        plsc.subcore_barrier()
        c0 = cid * per_core + sid * per_sub

        @pl.loop(0, per_sub)
        def _(i):
            pltpu.sync_copy(eidx_h.at[pl.ds(c0 + i, 1)], ebuf)
            pltpu.async_copy(table_h.at[ebuf.at[0, 0]], rows, sem).wait()
            pltpu.sync_copy(rows, acc.at[ebuf.at[0, 1]], add=True)

        plsc.subcore_barrier()

        @pl.when(cid == 0)
        def _():
            pltpu.sync_copy(acc.at[pl.ds(r0, rps)], out_h.at[0].at[pl.ds(r0, rps)])

        @pl.when(cid == 1)
        def _():
            pltpu.sync_copy(acc.at[pl.ds(r0, rps)], out_h.at[1].at[pl.ds(r0, rps)])

    return k(table, eidx, zrows)


def _sc_h_comb(tj0, tj1, tg0, tg1, e_jj, e_gj, zrows):
    rps = _NJP // 16

    @functools.partial(
        pl.kernel,
        out_type=[jax.ShapeDtypeStruct((_NJP, 64), _BF16)] * 4,
        mesh=plsc.VectorSubcoreMesh(**_MESH),
        scratch_types=[
            pltpu.VMEM((1, 2, _CH), jnp.int32),
            pltpu.VMEM((_CH, 64), _BF16),
            pltpu.VMEM_SHARED((_NJP, 64), _BF16),
            pltpu.SemaphoreType.DMA,
        ],
        compiler_params=_SC_PARAMS,
    )
    def k(tj0_h, tj1_h, tg0_h, tg1_h, ejj_h, egj_h, z_h,
          ojj0_h, ojj1_h, ogj0_h, ogj1_h, ebuf, rows, acc, sem):
        cid = lax.axis_index("c")
        sid = lax.axis_index("s")
        r0 = sid * rps

        def phase(t0_h, t1_h, eidx_h, nch, o0_h, o1_h):
            per_sub = nch // 16
            c0 = sid * per_sub
            pltpu.sync_copy(z_h.at[pl.ds(0, rps)], acc.at[pl.ds(r0, rps)])
            plsc.subcore_barrier()

            def body(t_h):
                @pl.loop(0, per_sub)
                def _(i):
                    pltpu.sync_copy(eidx_h.at[pl.ds(c0 + i, 1)], ebuf)
                    pltpu.async_copy(t_h.at[ebuf.at[0, 0]], rows, sem).wait()
                    pltpu.sync_copy(rows, acc.at[ebuf.at[0, 1]], add=True)

            @pl.when(cid == 0)
            def _():
                body(t0_h)

            @pl.when(cid == 1)
            def _():
                body(t1_h)

            plsc.subcore_barrier()

            @pl.when(cid == 0)
            def _():
                pltpu.sync_copy(acc.at[pl.ds(r0, rps)], o0_h.at[pl.ds(r0, rps)])

            @pl.when(cid == 1)
            def _():
                pltpu.sync_copy(acc.at[pl.ds(r0, rps)], o1_h.at[pl.ds(r0, rps)])

        phase(tj0_h, tj1_h, ejj_h, _NCH_JJ, ojj0_h, ojj1_h)
        phase(tg0_h, tg1_h, egj_h, _NCH_GJ, ogj0_h, ogj1_h)

    return k(tj0, tj1, tg0, tg1, e_jj, e_gj, zrows)


def _dot(a, b):
    return lax.dot_general(a, b, (((1,), (0,)), ((), ())),
                           precision=lax.Precision.DEFAULT,
                           preferred_element_type=_F32)


def _lstm(z, c_prev):
    ig = jax.nn.sigmoid(z[:, :64])
    fg = jax.nn.sigmoid(z[:, 64:128])
    gg = jnp.tanh(z[:, 128:192])
    og = jax.nn.sigmoid(z[:, 192:256])
    c = ig * gg if c_prev is None else fg * c_prev + ig * gg
    return og * jnp.tanh(c), c


def _mean16(p0, p1, col):
    s = p0 + p1
    return s * (1.0 / jnp.maximum(s[:, col:col + 1], 1.0))


def _split_dot(a0, a1, w):
    return _dot(a0, w[:32]) + _dot(a1, w[32:])


def _blk(width):
    return pl.BlockSpec((_B, width), lambda i: (i, 0))


def _full(a):
    return pl.BlockSpec(a.shape, lambda i: (0, 0))


def _tc_cell0(n_pad, xp, aggs, cols, weights):
    grid = (n_pad // _B,)
    na = len(aggs)
    arrays = [xp] + [p for a in aggs for p in a] + list(weights)
    specs = ([_blk(16)] * (1 + 2 * na) + [_full(w) for w in weights])

    def body(*refs):
        xr = refs[0]
        ar = refs[1:1 + 2 * na]
        wr = refs[1 + 2 * na:-2]
        t0o, t1o = refs[-2], refs[-1]
        x = xr[...]
        means = [_mean16(ar[2 * i][...], ar[2 * i + 1][...], cols[i])
                 for i in range(na)]
        it = iter(wr)
        ze = _dot(x, next(it)[...])
        for m in means:
            ze = ze + _dot(m, next(it)[...])
        ze = ze + next(it)[...]
        hE, cE = _lstm(ze, None)
        zd = _dot(x, next(it)[...]) + _dot(hE, next(it)[...])
        for m in means:
            zd = zd + _dot(m, next(it)[...])
        zd = zd + next(it)[...]
        hD, _ = _lstm(zd, cE)
        t0o[...] = jnp.concatenate([hE[:, :32], hD[:, :32]], 1).astype(_BF16)
        t1o[...] = jnp.concatenate([hE[:, 32:], hD[:, 32:]], 1).astype(_BF16)

    out_shape = [jax.ShapeDtypeStruct((n_pad, 64), _BF16)] * 2
    out_specs = [_blk(64)] * 2
    return pl.pallas_call(body, grid=grid, in_specs=specs,
                          out_specs=out_specs, out_shape=out_shape)(*arrays)


def _tc_final(tj0, tj1, jjA, gjA, rjj, rgj, xp, weights, whead):
    grid = (_NJP // _B,)
    arrays = ([tj0, tj1, jjA[0], jjA[1], gjA[0], gjA[1],
               rjj[0], rjj[1], rgj[0], rgj[1], xp]
              + list(weights) + [whead])
    specs = ([_blk(64)] * 6 + [_blk(16)] * 5
             + [_full(w) for w in weights] + [_full(whead)])

    def body(tj0r, tj1r, jjA0r, jjA1r, gjA0r, gjA1r,
             rjj0r, rjj1r, rgj0r, rgj1r, xr,
             we_x, we_jj, we_gj, be, wd_x, wd_h, wd_jj, wd_gj, bd,
             whr, oref):
        t0 = tj0r[...].astype(_F32)
        t1 = tj1r[...].astype(_F32)
        jA0 = jjA0r[...].astype(_F32)
        jA1 = jjA1r[...].astype(_F32)
        gA0 = gjA0r[...].astype(_F32)
        gA1 = gjA1r[...].astype(_F32)
        x = xr[...]
        sjj = rjj0r[...] + rjj1r[...]
        r_jj = 1.0 / jnp.maximum(sjj[:, 5:6], 1.0)
        sgj = rgj0r[...] + rgj1r[...]
        r_gj = 1.0 / jnp.maximum(sgj[:, 8:9], 1.0)

        ze = (_split_dot(t0[:, :32], t1[:, :32], we_x[...])
              + _split_dot(jA0[:, :32] * r_jj, jA1[:, :32] * r_jj, we_jj[...])
              + _split_dot(gA0[:, :32] * r_gj, gA1[:, :32] * r_gj, we_gj[...])
              + be[...])
        hE, cE = _lstm(ze, None)
        zd = (_split_dot(t0[:, 32:], t1[:, 32:], wd_x[...])
              + _dot(hE, wd_h[...])
              + _split_dot(jA0[:, 32:] * r_jj, jA1[:, 32:] * r_jj, wd_jj[...])
              + _split_dot(gA0[:, 32:] * r_gj, gA1[:, 32:] * r_gj, wd_gj[...])
              + bd[...])
        hD, cD = _lstm(zd, cE)
        wh = whr[...]
        zh = _dot(hD, wh[:64]) + _dot(cD, wh[64:128]) + _dot(x, wh[128:144])
        lane = lax.broadcasted_iota(jnp.int32, zh.shape, 1)
        full = jnp.where(lane < 2, jnp.tanh(zh) / 5.0, jax.nn.sigmoid(zh))
        oref[...] = full[:, :3]

    out_shape = jax.ShapeDtypeStruct((_NJP, 3), _F32)
    out_specs = pl.BlockSpec((_B, 3), lambda i: (i, 0))
    return pl.pallas_call(body, grid=grid, in_specs=specs,
                          out_specs=out_specs, out_shape=out_shape)(*arrays)


def _prep_edges(ei, nch, dst_pad):
    e = ei.astype(jnp.int32)
    pad = nch * _CH - e.shape[1]
    src = jnp.concatenate([e[0], jnp.zeros((pad,), jnp.int32)])
    dst = jnp.concatenate([e[1], jnp.full((pad,), dst_pad, jnp.int32)])
    return jnp.stack([src.reshape(nch, _CH), dst.reshape(nch, _CH)], axis=1)


def kernel(x_joint, x_grain, params, edge_jj, edge_gj, edge_jg, edge_attr_jj):
    p = params

    def wcat(stem, tail, pad_to=None):
        w = jnp.concatenate([p[f"{stem}_{g}_{tail}"] for g in _GATES], 1)
        if pad_to is not None and w.shape[0] < pad_to:
            w = jnp.zeros((pad_to, w.shape[1]), _F32).at[:w.shape[0]].set(w)
        return w

    def bcat(pre, nt):
        return jnp.concatenate(
            [p[f"{pre}_b_{g}_{nt}"] for g in _GATES]).reshape(1, 256)

    joint_pad = (jnp.zeros((_NJP, 16), _F32)
                 .at[:_NJ, :5].set(x_joint).at[:_NJ, 5].set(1.0))
    grain_pad = (jnp.zeros((_NGP, 16), _F32)
                 .at[:_NG, :8].set(x_grain).at[:_NG, 8].set(1.0))

    e_jj = _prep_edges(edge_jj, _NCH_JJ, _NJP - 1)
    e_gj = _prep_edges(edge_gj, _NCH_GJ, _NJP - 1)
    e_jg = _prep_edges(edge_jg, _NCH_GJ, _NGP - 1)

    z16 = jnp.zeros((_NJP // 16, 16), _F32)
    z64 = jnp.zeros((_NJP // 16, 64), _BF16)

    raw_jj = _sc_raw_agg(joint_pad, e_jj, z16, _NJP, _NCH_JJ)
    raw_gj = _sc_raw_agg(grain_pad, e_gj, z16, _NJP, _NCH_GJ)
    raw_jg = _sc_raw_agg(joint_pad, e_jg, z16[:_NGP // 16], _NGP, _NCH_GJ)
    rjj = [raw_jj[0], raw_jj[1]]
    rgj = [raw_gj[0], raw_gj[1]]
    rjg = [raw_jg[0], raw_jg[1]]

    tj0, tj1 = _tc_cell0(
        _NJP, joint_pad, [rjj, rgj], [5, 8],
        (wcat("enc0_Wx", "joint", 16),
         wcat("enc0_We", "jj", 16), wcat("enc0_We", "gj", 16),
         bcat("enc0", "joint"),
         wcat("dec0_Wx", "joint", 16), wcat("dec0_Wh", "joint"),
         wcat("dec0_We", "jj", 16), wcat("dec0_We", "gj", 16),
         bcat("dec0", "joint")))
    tg0, tg1 = _tc_cell0(
        _NGP, grain_pad, [rjg], [5],
        (wcat("enc0_Wx", "grain", 16),
         wcat("enc0_We", "jg", 16),
         bcat("enc0", "grain"),
         wcat("dec0_Wx", "grain", 16), wcat("dec0_Wh", "grain"),
         wcat("dec0_We", "jg", 16),
         bcat("dec0", "grain")))

    jjA0, jjA1, gjA0, gjA1 = _sc_h_comb(tj0, tj1, tg0, tg1, e_jj, e_gj, z64)

    whead = jnp.zeros((144, 128), _F32)
    whead = whead.at[:129, 0:2].set(p["lin1_W"])
    whead = whead.at[:129, 2:3].set(p["lin2_W"])
    whead = whead.at[133, 0:2].set(p["lin1_b"])
    whead = whead.at[133, 2].set(p["lin2_b"][0])

    out = _tc_final(
        tj0, tj1, [jjA0, jjA1], [gjA0, gjA1], rjj, rgj, joint_pad,
        (wcat("enc1_Wx", "joint"),
         wcat("enc1_We", "jj"), wcat("enc1_We", "gj"),
         bcat("enc1", "joint"),
         wcat("dec1_Wx", "joint"), wcat("dec1_Wh", "joint"),
         wcat("dec1_We", "jj"), wcat("dec1_We", "gj"),
         bcat("dec1", "joint")),
        whead)

    return out[:_NJ]


# speedup vs baseline: 1.5635x; 1.0052x over previous
import functools

import jax
import jax.numpy as jnp
from jax import lax
from jax.experimental import pallas as pl
from jax.experimental.pallas import tpu as pltpu
from jax.experimental.pallas import tpu_sc as plsc

_NJ, _NG, _C = 50000, 25000, 64
_NJP, _NGP = 50176, 25088
_CH = 512
_NCH_JJ = 1600
_NCH_GJ = 320

_GATES = ("i", "f", "g", "o")
_B = 512
_F32 = jnp.float32
_BF16 = jnp.bfloat16

_MESH = dict(core_axis_name="c", subcore_axis_name="s")
_SC_PARAMS = pltpu.CompilerParams(use_tc_tiling_on_sc=False)


def _sc_raw_agg(table, eidx, zrows, n_dst, nch):
    per_core = nch // 2
    per_sub = per_core // 16
    rps = n_dst // 16

    @functools.partial(
        pl.kernel,
        out_type=jax.ShapeDtypeStruct((2, n_dst, 16), _F32),
        mesh=plsc.VectorSubcoreMesh(**_MESH),
        scratch_types=[
            pltpu.VMEM((1, 2, _CH), jnp.int32),
            pltpu.VMEM((_CH, 16), _F32),
            pltpu.VMEM_SHARED((n_dst, 16), _F32),
            pltpu.SemaphoreType.DMA,
        ],
        compiler_params=_SC_PARAMS,
    )
    def k(table_h, eidx_h, z_h, out_h, ebuf, rows, acc, sem):
        cid = lax.axis_index("c")
        sid = lax.axis_index("s")
        r0 = sid * rps
        pltpu.sync_copy(z_h.at[pl.ds(0, rps)], acc.at[pl.ds(r0, rps)])
        plsc.subcore_barrier()
        c0 = cid * per_core + sid * per_sub

        @pl.loop(0, per_sub)
        def _(i):
            pltpu.sync_copy(eidx_h.at[pl.ds(c0 + i, 1)], ebuf)
            pltpu.async_copy(table_h.at[ebuf.at[0, 0]], rows, sem).wait()
            pltpu.sync_copy(rows, acc.at[ebuf.at[0, 1]], add=True)

        plsc.subcore_barrier()

        @pl.when(cid == 0)
        def _():
            pltpu.sync_copy(acc.at[pl.ds(r0, rps)], out_h.at[0].at[pl.ds(r0, rps)])

        @pl.when(cid == 1)
        def _():
            pltpu.sync_copy(acc.at[pl.ds(r0, rps)], out_h.at[1].at[pl.ds(r0, rps)])

    return k(table, eidx, zrows)


def _sc_h_comb(tj0, tj1, tg0, tg1, e_jj, e_gj, zrows):
    rps = _NJP // 16

    @functools.partial(
        pl.kernel,
        out_type=[jax.ShapeDtypeStruct((_NJP, 64), _BF16)] * 4,
        mesh=plsc.VectorSubcoreMesh(**_MESH),
        scratch_types=[
            pltpu.VMEM((1, 2, _CH), jnp.int32),
            pltpu.VMEM((_CH, 64), _BF16),
            pltpu.VMEM_SHARED((_NJP, 64), _BF16),
            pltpu.SemaphoreType.DMA,
        ],
        compiler_params=_SC_PARAMS,
    )
    def k(tj0_h, tj1_h, tg0_h, tg1_h, ejj_h, egj_h, z_h,
          ojj0_h, ojj1_h, ogj0_h, ogj1_h, ebuf, rows, acc, sem):
        cid = lax.axis_index("c")
        sid = lax.axis_index("s")
        r0 = sid * rps

        def phase(t0_h, t1_h, eidx_h, nch, o0_h, o1_h):
            per_sub = nch // 16
            c0 = sid * per_sub
            pltpu.sync_copy(z_h.at[pl.ds(0, rps)], acc.at[pl.ds(r0, rps)])
            plsc.subcore_barrier()

            def body(t_h):
                @pl.loop(0, per_sub)
                def _(i):
                    pltpu.sync_copy(eidx_h.at[pl.ds(c0 + i, 1)], ebuf)
                    pltpu.async_copy(t_h.at[ebuf.at[0, 0]], rows, sem).wait()
                    pltpu.sync_copy(rows, acc.at[ebuf.at[0, 1]], add=True)

            @pl.when(cid == 0)
            def _():
                body(t0_h)

            @pl.when(cid == 1)
            def _():
                body(t1_h)

            plsc.subcore_barrier()

            @pl.when(cid == 0)
            def _():
                pltpu.sync_copy(acc.at[pl.ds(r0, rps)], o0_h.at[pl.ds(r0, rps)])

            @pl.when(cid == 1)
            def _():
                pltpu.sync_copy(acc.at[pl.ds(r0, rps)], o1_h.at[pl.ds(r0, rps)])

        phase(tj0_h, tj1_h, ejj_h, _NCH_JJ, ojj0_h, ojj1_h)
        phase(tg0_h, tg1_h, egj_h, _NCH_GJ, ogj0_h, ogj1_h)

    return k(tj0, tj1, tg0, tg1, e_jj, e_gj, zrows)


def _dot(a, b):
    return lax.dot_general(a, b, (((1,), (0,)), ((), ())),
                           precision=lax.Precision.DEFAULT,
                           preferred_element_type=_F32)


def _lstm(z, c_prev):
    ig = jax.nn.sigmoid(z[:, :64])
    fg = jax.nn.sigmoid(z[:, 64:128])
    gg = jnp.tanh(z[:, 128:192])
    og = jax.nn.sigmoid(z[:, 192:256])
    c = ig * gg if c_prev is None else fg * c_prev + ig * gg
    return og * jnp.tanh(c), c


def _mean16(p0, p1, col):
    s = p0 + p1
    return s * (1.0 / jnp.maximum(s[:, col:col + 1], 1.0))


def _split_dot(a0, a1, w):
    return _dot(a0, w[:32]) + _dot(a1, w[32:])


def _blk(width):
    return pl.BlockSpec((_B, width), lambda i: (i, 0))


def _full(a):
    return pl.BlockSpec(a.shape, lambda i: (0, 0))


def _tc_cell0(n_pad, xp, aggs, cols, weights):
    grid = (n_pad // _B,)
    na = len(aggs)
    arrays = [xp] + [p for a in aggs for p in a] + list(weights)
    specs = ([_blk(16)] * (1 + 2 * na) + [_full(w) for w in weights])

    def body(*refs):
        xr = refs[0]
        ar = refs[1:1 + 2 * na]
        wr = refs[1 + 2 * na:-2]
        t0o, t1o = refs[-2], refs[-1]
        x = xr[...]
        means = [_mean16(ar[2 * i][...], ar[2 * i + 1][...], cols[i])
                 for i in range(na)]
        it = iter(wr)
        ze = _dot(x, next(it)[...])
        for m in means:
            ze = ze + _dot(m, next(it)[...])
        ze = ze + next(it)[...]
        hE, cE = _lstm(ze, None)
        zd = _dot(x, next(it)[...]) + _dot(hE, next(it)[...])
        for m in means:
            zd = zd + _dot(m, next(it)[...])
        zd = zd + next(it)[...]
        hD, _ = _lstm(zd, cE)
        t0o[...] = jnp.concatenate([hE[:, :32], hD[:, :32]], 1).astype(_BF16)
        t1o[...] = jnp.concatenate([hE[:, 32:], hD[:, 32:]], 1).astype(_BF16)

    out_shape = [jax.ShapeDtypeStruct((n_pad, 64), _BF16)] * 2
    out_specs = [_blk(64)] * 2
    return pl.pallas_call(body, grid=grid, in_specs=specs,
                          out_specs=out_specs, out_shape=out_shape)(*arrays)


def _tc_final(tj0, tj1, jjA, gjA, rjj, rgj, xp, weights, whead):
    grid = (_NJP // _B,)
    arrays = ([tj0, tj1, jjA[0], jjA[1], gjA[0], gjA[1],
               rjj[0], rjj[1], rgj[0], rgj[1], xp]
              + list(weights) + [whead])
    specs = ([_blk(64)] * 6 + [_blk(16)] * 5
             + [_full(w) for w in weights] + [_full(whead)])

    def body(tj0r, tj1r, jjA0r, jjA1r, gjA0r, gjA1r,
             rjj0r, rjj1r, rgj0r, rgj1r, xr,
             we_x, we_jj, we_gj, be, wd_x, wd_h, wd_jj, wd_gj, bd,
             whr, oref):
        t0 = tj0r[...].astype(_F32)
        t1 = tj1r[...].astype(_F32)
        jA0 = jjA0r[...].astype(_F32)
        jA1 = jjA1r[...].astype(_F32)
        gA0 = gjA0r[...].astype(_F32)
        gA1 = gjA1r[...].astype(_F32)
        x = xr[...]
        sjj = rjj0r[...] + rjj1r[...]
        r_jj = 1.0 / jnp.maximum(sjj[:, 5:6], 1.0)
        sgj = rgj0r[...] + rgj1r[...]
        r_gj = 1.0 / jnp.maximum(sgj[:, 8:9], 1.0)

        ze = (_split_dot(t0[:, :32], t1[:, :32], we_x[...])
              + _split_dot(jA0[:, :32] * r_jj, jA1[:, :32] * r_jj, we_jj[...])
              + _split_dot(gA0[:, :32] * r_gj, gA1[:, :32] * r_gj, we_gj[...])
              + be[...])
        hE, cE = _lstm(ze, None)
        zd = (_split_dot(t0[:, 32:], t1[:, 32:], wd_x[...])
              + _dot(hE, wd_h[...])
              + _split_dot(jA0[:, 32:] * r_jj, jA1[:, 32:] * r_jj, wd_jj[...])
              + _split_dot(gA0[:, 32:] * r_gj, gA1[:, 32:] * r_gj, wd_gj[...])
              + bd[...])
        hD, cD = _lstm(zd, cE)
        wh = whr[...]
        zh = _dot(hD, wh[:64]) + _dot(cD, wh[64:128]) + _dot(x, wh[128:144])
        lane = lax.broadcasted_iota(jnp.int32, zh.shape, 1)
        full = jnp.where(lane < 2, jnp.tanh(zh) / 5.0, jax.nn.sigmoid(zh))
        oref[...] = full[:, :3]

    out_shape = jax.ShapeDtypeStruct((_NJP, 3), _F32)
    out_specs = pl.BlockSpec((_B, 3), lambda i: (i, 0))
    return pl.pallas_call(body, grid=grid, in_specs=specs,
                          out_specs=out_specs, out_shape=out_shape)(*arrays)


def _prep_edges(ei, nch, n_real, n_pad):
    e = ei.astype(jnp.int32)
    pad = nch * _CH - e.shape[1]
    src = jnp.concatenate([e[0], jnp.zeros((pad,), jnp.int32)])
    pad_dst = n_real + jnp.arange(pad, dtype=jnp.int32) % (n_pad - n_real)
    dst = jnp.concatenate([e[1], pad_dst])
    return jnp.stack([src.reshape(nch, _CH), dst.reshape(nch, _CH)], axis=1)


def kernel(x_joint, x_grain, params, edge_jj, edge_gj, edge_jg, edge_attr_jj):
    p = params

    def wcat(stem, tail, pad_to=None):
        w = jnp.concatenate([p[f"{stem}_{g}_{tail}"] for g in _GATES], 1)
        if pad_to is not None and w.shape[0] < pad_to:
            w = jnp.zeros((pad_to, w.shape[1]), _F32).at[:w.shape[0]].set(w)
        return w

    def bcat(pre, nt):
        return jnp.concatenate(
            [p[f"{pre}_b_{g}_{nt}"] for g in _GATES]).reshape(1, 256)

    joint_pad = (jnp.zeros((_NJP, 16), _F32)
                 .at[:_NJ, :5].set(x_joint).at[:_NJ, 5].set(1.0))
    grain_pad = (jnp.zeros((_NGP, 16), _F32)
                 .at[:_NG, :8].set(x_grain).at[:_NG, 8].set(1.0))

    e_jj = _prep_edges(edge_jj, _NCH_JJ, _NJ, _NJP)
    e_gj = _prep_edges(edge_gj, _NCH_GJ, _NJ, _NJP)
    e_jg = _prep_edges(edge_jg, _NCH_GJ, _NG, _NGP)

    z16 = jnp.zeros((_NJP // 16, 16), _F32)
    z64 = jnp.zeros((_NJP // 16, 64), _BF16)

    raw_jj = _sc_raw_agg(joint_pad, e_jj, z16, _NJP, _NCH_JJ)
    raw_gj = _sc_raw_agg(grain_pad, e_gj, z16, _NJP, _NCH_GJ)
    raw_jg = _sc_raw_agg(joint_pad, e_jg, z16[:_NGP // 16], _NGP, _NCH_GJ)
    rjj = [raw_jj[0], raw_jj[1]]
    rgj = [raw_gj[0], raw_gj[1]]
    rjg = [raw_jg[0], raw_jg[1]]

    tj0, tj1 = _tc_cell0(
        _NJP, joint_pad, [rjj, rgj], [5, 8],
        (wcat("enc0_Wx", "joint", 16),
         wcat("enc0_We", "jj", 16), wcat("enc0_We", "gj", 16),
         bcat("enc0", "joint"),
         wcat("dec0_Wx", "joint", 16), wcat("dec0_Wh", "joint"),
         wcat("dec0_We", "jj", 16), wcat("dec0_We", "gj", 16),
         bcat("dec0", "joint")))
    tg0, tg1 = _tc_cell0(
        _NGP, grain_pad, [rjg], [5],
        (wcat("enc0_Wx", "grain", 16),
         wcat("enc0_We", "jg", 16),
         bcat("enc0", "grain"),
         wcat("dec0_Wx", "grain", 16), wcat("dec0_Wh", "grain"),
         wcat("dec0_We", "jg", 16),
         bcat("dec0", "grain")))

    jjA0, jjA1, gjA0, gjA1 = _sc_h_comb(tj0, tj1, tg0, tg1, e_jj, e_gj, z64)

    whead = jnp.zeros((144, 128), _F32)
    whead = whead.at[:129, 0:2].set(p["lin1_W"])
    whead = whead.at[:129, 2:3].set(p["lin2_W"])
    whead = whead.at[133, 0:2].set(p["lin1_b"])
    whead = whead.at[133, 2].set(p["lin2_b"][0])

    out = _tc_final(
        tj0, tj1, [jjA0, jjA1], [gjA0, gjA1], rjj, rgj, joint_pad,
        (wcat("enc1_Wx", "joint"),
         wcat("enc1_We", "jj"), wcat("enc1_We", "gj"),
         bcat("enc1", "joint"),
         wcat("dec1_Wx", "joint"), wcat("dec1_Wh", "joint"),
         wcat("dec1_We", "jj"), wcat("dec1_We", "gj"),
         bcat("dec1", "joint")),
        whead)

    return out[:_NJ]


# speedup vs baseline: 1.5797x; 1.0104x over previous
import functools

import jax
import jax.numpy as jnp
from jax import lax
from jax.experimental import pallas as pl
from jax.experimental.pallas import tpu as pltpu
from jax.experimental.pallas import tpu_sc as plsc

_NJ, _NG, _C = 50000, 25000, 64
_NJP, _NGP = 50176, 25088
_CH = 512
_NCH_JJ = 1600
_NCH_GJ = 320

_GATES = ("i", "f", "g", "o")
_B = 512
_F32 = jnp.float32
_BF16 = jnp.bfloat16

_MESH = dict(core_axis_name="c", subcore_axis_name="s")
_SC_PARAMS = pltpu.CompilerParams(use_tc_tiling_on_sc=False)


def _sc_raw_agg(table, eidx, zrows, n_dst, nch):
    per_core = nch // 2
    per_sub = per_core // 16
    rps = n_dst // 16

    @functools.partial(
        pl.kernel,
        out_type=jax.ShapeDtypeStruct((2, n_dst, 16), _F32),
        mesh=plsc.VectorSubcoreMesh(**_MESH),
        scratch_types=[
            pltpu.VMEM((1, 2, _CH), jnp.int32),
            pltpu.VMEM((1, 2, _CH), jnp.int32),
            pltpu.VMEM((_CH, 16), _F32),
            pltpu.VMEM((_CH, 16), _F32),
            pltpu.VMEM_SHARED((n_dst, 16), _F32),
            pltpu.SemaphoreType.DMA,
            pltpu.SemaphoreType.DMA,
        ],
        compiler_params=_SC_PARAMS,
    )
    def k(table_h, eidx_h, z_h, out_h, ebuf_a, ebuf_b, rows_a, rows_b,
          acc, sem_a, sem_b):
        cid = lax.axis_index("c")
        sid = lax.axis_index("s")
        r0 = sid * rps
        pltpu.sync_copy(z_h.at[pl.ds(0, rps)], acc.at[pl.ds(r0, rps)])
        plsc.subcore_barrier()
        c0 = cid * per_core + sid * per_sub

        @pl.loop(0, per_sub // 2)
        def _(j):
            i0 = c0 + 2 * j
            pltpu.sync_copy(eidx_h.at[pl.ds(i0, 1)], ebuf_a)
            cp_a = pltpu.async_copy(table_h.at[ebuf_a.at[0, 0]], rows_a, sem_a)
            pltpu.sync_copy(eidx_h.at[pl.ds(i0 + 1, 1)], ebuf_b)
            cp_b = pltpu.async_copy(table_h.at[ebuf_b.at[0, 0]], rows_b, sem_b)
            cp_a.wait()
            pltpu.sync_copy(rows_a, acc.at[ebuf_a.at[0, 1]], add=True)
            cp_b.wait()
            pltpu.sync_copy(rows_b, acc.at[ebuf_b.at[0, 1]], add=True)

        plsc.subcore_barrier()

        @pl.when(cid == 0)
        def _():
            pltpu.sync_copy(acc.at[pl.ds(r0, rps)], out_h.at[0].at[pl.ds(r0, rps)])

        @pl.when(cid == 1)
        def _():
            pltpu.sync_copy(acc.at[pl.ds(r0, rps)], out_h.at[1].at[pl.ds(r0, rps)])

    return k(table, eidx, zrows)


def _sc_h_comb(tj0, tj1, tg0, tg1, e_jj, e_gj, zrows):
    rps = _NJP // 16

    @functools.partial(
        pl.kernel,
        out_type=[jax.ShapeDtypeStruct((_NJP, 64), _BF16)] * 4,
        mesh=plsc.VectorSubcoreMesh(**_MESH),
        scratch_types=[
            pltpu.VMEM((1, 2, _CH), jnp.int32),
            pltpu.VMEM((_CH // 2, 64), _BF16),
            pltpu.VMEM((_CH // 2, 64), _BF16),
            pltpu.VMEM_SHARED((_NJP, 64), _BF16),
            pltpu.SemaphoreType.DMA,
            pltpu.SemaphoreType.DMA,
        ],
        compiler_params=_SC_PARAMS,
    )
    def k(tj0_h, tj1_h, tg0_h, tg1_h, ejj_h, egj_h, z_h,
          ojj0_h, ojj1_h, ogj0_h, ogj1_h, ebuf, rows_a, rows_b,
          acc, sem_a, sem_b):
        cid = lax.axis_index("c")
        sid = lax.axis_index("s")
        r0 = sid * rps

        def phase(t0_h, t1_h, eidx_h, nch, o0_h, o1_h):
            per_sub = nch // 16
            c0 = sid * per_sub
            pltpu.sync_copy(z_h.at[pl.ds(0, rps)], acc.at[pl.ds(r0, rps)])
            plsc.subcore_barrier()

            def body(t_h):
                hc = _CH // 2

                @pl.loop(0, per_sub)
                def _(i):
                    pltpu.sync_copy(eidx_h.at[pl.ds(c0 + i, 1)], ebuf)
                    cp_a = pltpu.async_copy(
                        t_h.at[ebuf.at[0, 0, pl.ds(0, hc)]], rows_a, sem_a)
                    cp_b = pltpu.async_copy(
                        t_h.at[ebuf.at[0, 0, pl.ds(hc, hc)]], rows_b, sem_b)
                    cp_a.wait()
                    pltpu.sync_copy(
                        rows_a, acc.at[ebuf.at[0, 1, pl.ds(0, hc)]], add=True)
                    cp_b.wait()
                    pltpu.sync_copy(
                        rows_b, acc.at[ebuf.at[0, 1, pl.ds(hc, hc)]], add=True)

            @pl.when(cid == 0)
            def _():
                body(t0_h)

            @pl.when(cid == 1)
            def _():
                body(t1_h)

            plsc.subcore_barrier()

            @pl.when(cid == 0)
            def _():
                pltpu.sync_copy(acc.at[pl.ds(r0, rps)], o0_h.at[pl.ds(r0, rps)])

            @pl.when(cid == 1)
            def _():
                pltpu.sync_copy(acc.at[pl.ds(r0, rps)], o1_h.at[pl.ds(r0, rps)])

        phase(tj0_h, tj1_h, ejj_h, _NCH_JJ, ojj0_h, ojj1_h)
        phase(tg0_h, tg1_h, egj_h, _NCH_GJ, ogj0_h, ogj1_h)

    return k(tj0, tj1, tg0, tg1, e_jj, e_gj, zrows)


def _dot(a, b):
    return lax.dot_general(a, b, (((1,), (0,)), ((), ())),
                           precision=lax.Precision.DEFAULT,
                           preferred_element_type=_F32)


def _lstm(z, c_prev):
    ig = jax.nn.sigmoid(z[:, :64])
    fg = jax.nn.sigmoid(z[:, 64:128])
    gg = jnp.tanh(z[:, 128:192])
    og = jax.nn.sigmoid(z[:, 192:256])
    c = ig * gg if c_prev is None else fg * c_prev + ig * gg
    return og * jnp.tanh(c), c


def _mean16(p0, p1, col):
    s = p0 + p1
    return s * (1.0 / jnp.maximum(s[:, col:col + 1], 1.0))


def _split_dot(a0, a1, w):
    return _dot(a0, w[:32]) + _dot(a1, w[32:])


def _blk(width):
    return pl.BlockSpec((_B, width), lambda i: (i, 0))


def _full(a):
    return pl.BlockSpec(a.shape, lambda i: (0, 0))


def _tc_cell0(n_pad, xp, aggs, cols, weights):
    grid = (n_pad // _B,)
    na = len(aggs)
    arrays = [xp] + [p for a in aggs for p in a] + list(weights)
    specs = ([_blk(16)] * (1 + 2 * na) + [_full(w) for w in weights])

    def body(*refs):
        xr = refs[0]
        ar = refs[1:1 + 2 * na]
        wr = refs[1 + 2 * na:-2]
        t0o, t1o = refs[-2], refs[-1]
        x = xr[...]
        means = [_mean16(ar[2 * i][...], ar[2 * i + 1][...], cols[i])
                 for i in range(na)]
        it = iter(wr)
        ze = _dot(x, next(it)[...])
        for m in means:
            ze = ze + _dot(m, next(it)[...])
        ze = ze + next(it)[...]
        hE, cE = _lstm(ze, None)
        zd = _dot(x, next(it)[...]) + _dot(hE, next(it)[...])
        for m in means:
            zd = zd + _dot(m, next(it)[...])
        zd = zd + next(it)[...]
        hD, _ = _lstm(zd, cE)
        t0o[...] = jnp.concatenate([hE[:, :32], hD[:, :32]], 1).astype(_BF16)
        t1o[...] = jnp.concatenate([hE[:, 32:], hD[:, 32:]], 1).astype(_BF16)

    out_shape = [jax.ShapeDtypeStruct((n_pad, 64), _BF16)] * 2
    out_specs = [_blk(64)] * 2
    return pl.pallas_call(body, grid=grid, in_specs=specs,
                          out_specs=out_specs, out_shape=out_shape)(*arrays)


def _tc_final(tj0, tj1, jjA, gjA, rjj, rgj, xp, weights, whead):
    grid = (_NJP // _B,)
    arrays = ([tj0, tj1, jjA[0], jjA[1], gjA[0], gjA[1],
               rjj[0], rjj[1], rgj[0], rgj[1], xp]
              + list(weights) + [whead])
    specs = ([_blk(64)] * 6 + [_blk(16)] * 5
             + [_full(w) for w in weights] + [_full(whead)])

    def body(tj0r, tj1r, jjA0r, jjA1r, gjA0r, gjA1r,
             rjj0r, rjj1r, rgj0r, rgj1r, xr,
             we_x, we_jj, we_gj, be, wd_x, wd_h, wd_jj, wd_gj, bd,
             whr, oref):
        t0 = tj0r[...].astype(_F32)
        t1 = tj1r[...].astype(_F32)
        jA0 = jjA0r[...].astype(_F32)
        jA1 = jjA1r[...].astype(_F32)
        gA0 = gjA0r[...].astype(_F32)
        gA1 = gjA1r[...].astype(_F32)
        x = xr[...]
        sjj = rjj0r[...] + rjj1r[...]
        r_jj = 1.0 / jnp.maximum(sjj[:, 5:6], 1.0)
        sgj = rgj0r[...] + rgj1r[...]
        r_gj = 1.0 / jnp.maximum(sgj[:, 8:9], 1.0)

        ze = (_split_dot(t0[:, :32], t1[:, :32], we_x[...])
              + _split_dot(jA0[:, :32] * r_jj, jA1[:, :32] * r_jj, we_jj[...])
              + _split_dot(gA0[:, :32] * r_gj, gA1[:, :32] * r_gj, we_gj[...])
              + be[...])
        hE, cE = _lstm(ze, None)
        zd = (_split_dot(t0[:, 32:], t1[:, 32:], wd_x[...])
              + _dot(hE, wd_h[...])
              + _split_dot(jA0[:, 32:] * r_jj, jA1[:, 32:] * r_jj, wd_jj[...])
              + _split_dot(gA0[:, 32:] * r_gj, gA1[:, 32:] * r_gj, wd_gj[...])
              + bd[...])
        hD, cD = _lstm(zd, cE)
        wh = whr[...]
        zh = _dot(hD, wh[:64]) + _dot(cD, wh[64:128]) + _dot(x, wh[128:144])
        lane = lax.broadcasted_iota(jnp.int32, zh.shape, 1)
        full = jnp.where(lane < 2, jnp.tanh(zh) / 5.0, jax.nn.sigmoid(zh))
        oref[...] = full[:, :3]

    out_shape = jax.ShapeDtypeStruct((_NJP, 3), _F32)
    out_specs = pl.BlockSpec((_B, 3), lambda i: (i, 0))
    return pl.pallas_call(body, grid=grid, in_specs=specs,
                          out_specs=out_specs, out_shape=out_shape)(*arrays)


def _prep_edges(ei, nch, n_real, n_pad):
    e = ei.astype(jnp.int32)
    pad = nch * _CH - e.shape[1]
    src = jnp.concatenate([e[0], jnp.zeros((pad,), jnp.int32)])
    pad_dst = n_real + jnp.arange(pad, dtype=jnp.int32) % (n_pad - n_real)
    dst = jnp.concatenate([e[1], pad_dst])
    return jnp.stack([src.reshape(nch, _CH), dst.reshape(nch, _CH)], axis=1)


def kernel(x_joint, x_grain, params, edge_jj, edge_gj, edge_jg, edge_attr_jj):
    p = params

    def wcat(stem, tail, pad_to=None):
        w = jnp.concatenate([p[f"{stem}_{g}_{tail}"] for g in _GATES], 1)
        if pad_to is not None and w.shape[0] < pad_to:
            w = jnp.zeros((pad_to, w.shape[1]), _F32).at[:w.shape[0]].set(w)
        return w

    def bcat(pre, nt):
        return jnp.concatenate(
            [p[f"{pre}_b_{g}_{nt}"] for g in _GATES]).reshape(1, 256)

    joint_pad = (jnp.zeros((_NJP, 16), _F32)
                 .at[:_NJ, :5].set(x_joint).at[:_NJ, 5].set(1.0))
    grain_pad = (jnp.zeros((_NGP, 16), _F32)
                 .at[:_NG, :8].set(x_grain).at[:_NG, 8].set(1.0))

    e_jj = _prep_edges(edge_jj, _NCH_JJ, _NJ, _NJP)
    e_gj = _prep_edges(edge_gj, _NCH_GJ, _NJ, _NJP)
    e_jg = _prep_edges(edge_jg, _NCH_GJ, _NG, _NGP)

    z16 = jnp.zeros((_NJP // 16, 16), _F32)
    z64 = jnp.zeros((_NJP // 16, 64), _BF16)

    raw_jj = _sc_raw_agg(joint_pad, e_jj, z16, _NJP, _NCH_JJ)
    raw_gj = _sc_raw_agg(grain_pad, e_gj, z16, _NJP, _NCH_GJ)
    raw_jg = _sc_raw_agg(joint_pad, e_jg, z16[:_NGP // 16], _NGP, _NCH_GJ)
    rjj = [raw_jj[0], raw_jj[1]]
    rgj = [raw_gj[0], raw_gj[1]]
    rjg = [raw_jg[0], raw_jg[1]]

    tj0, tj1 = _tc_cell0(
        _NJP, joint_pad, [rjj, rgj], [5, 8],
        (wcat("enc0_Wx", "joint", 16),
         wcat("enc0_We", "jj", 16), wcat("enc0_We", "gj", 16),
         bcat("enc0", "joint"),
         wcat("dec0_Wx", "joint", 16), wcat("dec0_Wh", "joint"),
         wcat("dec0_We", "jj", 16), wcat("dec0_We", "gj", 16),
         bcat("dec0", "joint")))
    tg0, tg1 = _tc_cell0(
        _NGP, grain_pad, [rjg], [5],
        (wcat("enc0_Wx", "grain", 16),
         wcat("enc0_We", "jg", 16),
         bcat("enc0", "grain"),
         wcat("dec0_Wx", "grain", 16), wcat("dec0_Wh", "grain"),
         wcat("dec0_We", "jg", 16),
         bcat("dec0", "grain")))

    jjA0, jjA1, gjA0, gjA1 = _sc_h_comb(tj0, tj1, tg0, tg1, e_jj, e_gj, z64)

    whead = jnp.zeros((144, 128), _F32)
    whead = whead.at[:129, 0:2].set(p["lin1_W"])
    whead = whead.at[:129, 2:3].set(p["lin2_W"])
    whead = whead.at[133, 0:2].set(p["lin1_b"])
    whead = whead.at[133, 2].set(p["lin2_b"][0])

    out = _tc_final(
        tj0, tj1, [jjA0, jjA1], [gjA0, gjA1], rjj, rgj, joint_pad,
        (wcat("enc1_Wx", "joint"),
         wcat("enc1_We", "jj"), wcat("enc1_We", "gj"),
         bcat("enc1", "joint"),
         wcat("dec1_Wx", "joint"), wcat("dec1_Wh", "joint"),
         wcat("dec1_We", "jj"), wcat("dec1_We", "gj"),
         bcat("dec1", "joint")),
        whead)

    return out[:_NJ]
